# baseline (speedup 1.0000x reference)
import functools

import jax
import jax.numpy as jnp
from jax import lax
from jax.experimental import pallas as pl
from jax.experimental.pallas import tpu as pltpu
from jax.experimental.pallas import tpu_sc as plsc

N = 10000
E = 160000
D = 256
C = 8
G = 16
GC = G * C


_BN = 2000
_GRID = N // _BN


def _tc1_body(x_ref, b_ref, ws1_ref, ws2_ref, wbc_ref, t_ref,
              m_ref, num_ref, den_ref, mx_ref):
    i = pl.program_id(0)
    f32 = jnp.float32
    bf16 = jnp.bfloat16
    x16 = x_ref[...].astype(bf16)
    h = jnp.maximum(jnp.dot(x16, ws1_ref[...], preferred_element_type=f32), 0.0)
    sc = jnp.dot(h.astype(bf16), ws2_ref[...], preferred_element_type=f32)
    lane = lax.broadcasted_iota(jnp.int32, sc.shape, 1)
    scm = jnp.where(lane < C, sc, jnp.float32(-1e30))
    scm = scm - jnp.max(scm, axis=1, keepdims=True)
    e = jnp.exp(scm)
    sm = e / jnp.sum(e, axis=1, keepdims=True)
    m_tiled = jnp.dot(sm.astype(bf16), t_ref[...], preferred_element_type=f32)
    oh_rep = (b_ref[...] == (lane // C)).astype(f32)
    M16 = (oh_rep * m_tiled).astype(bf16)
    m_ref[...] = M16

    h2 = jnp.dot(x16, wbc_ref[...], preferred_element_type=f32)
    dn = (((0,), (0,)), ((), ()))
    num_p = lax.dot_general(M16, h2.astype(bf16), dn, preferred_element_type=f32)
    den_p = lax.dot_general(M16, jnp.ones_like(h2, bf16), dn,
                            preferred_element_type=f32)
    mx_p = lax.dot_general(M16, x16, dn, preferred_element_type=f32)

    @pl.when(i == 0)
    def _():
        num_ref[...] = num_p
        den_ref[...] = den_p
        mx_ref[...] = mx_p

    @pl.when(i > 0)
    def _():
        num_ref[...] += num_p
        den_ref[...] += den_p
        mx_ref[...] += mx_p


def _tc1(x, batch_col, W_s1, W_s2p, W_bc, T):
    f32 = jnp.float32
    return pl.pallas_call(
        _tc1_body,
        grid=(_GRID,),
        in_specs=[
            pl.BlockSpec((_BN, D), lambda i: (i, 0)),
            pl.BlockSpec((_BN, 1), lambda i: (i, 0)),
            pl.BlockSpec((D, D), lambda i: (0, 0)),
            pl.BlockSpec((D, GC), lambda i: (0, 0)),
            pl.BlockSpec((D, D), lambda i: (0, 0)),
            pl.BlockSpec((GC, GC), lambda i: (0, 0)),
        ],
        out_specs=[
            pl.BlockSpec((_BN, GC), lambda i: (i, 0)),
            pl.BlockSpec((GC, D), lambda i: (0, 0)),
            pl.BlockSpec((GC, D), lambda i: (0, 0)),
            pl.BlockSpec((GC, D), lambda i: (0, 0)),
        ],
        out_shape=[
            jax.ShapeDtypeStruct((N, GC), jnp.bfloat16),
            jax.ShapeDtypeStruct((GC, D), f32),
            jax.ShapeDtypeStruct((GC, D), f32),
            jax.ShapeDtypeStruct((GC, D), f32),
        ],
    )(x, batch_col, W_s1, W_s2p, W_bc, T)


_DH = 128
_CHUNK = 128
_NCHUNK = 79
_EPT = _CHUNK * _NCHUNK
_EPAD = 16 * _EPT
_RPT = 632
_NPAD = 16 * _RPT


def _sc_body(xcat_hbm, src_hbm, dst_hbm, zeros_hbm, out_hbm,
             src_v, dst_v, rows_v, acc_sh, sem):
    c = lax.axis_index("c")
    s = lax.axis_index("s")
    pltpu.sync_copy(zeros_hbm, acc_sh.at[pl.ds(s * _RPT, _RPT)])
    pltpu.sync_copy(src_hbm.at[c, s], src_v)
    pltpu.sync_copy(dst_hbm.at[s], dst_v)
    plsc.subcore_barrier()

    def body(j, carry):
        pltpu.async_copy(xcat_hbm.at[src_v.at[j]], rows_v, sem).wait()
        pltpu.sync_copy(rows_v, acc_sh.at[dst_v.at[j]], add=True)
        return carry

    lax.fori_loop(0, _NCHUNK, body, 0)
    plsc.subcore_barrier()
    pltpu.sync_copy(acc_sh.at[pl.ds(s * _RPT, _RPT)],
                    out_hbm.at[c, pl.ds(s * _RPT, _RPT)])


def _sc_segment_sum(x, edge_index):
    f32 = jnp.float32
    i32 = jnp.int32
    src = edge_index[0].astype(i32)
    dst = edge_index[1].astype(i32)
    pad = _EPAD - E
    srcp = jnp.concatenate([src, jnp.zeros((pad,), i32)])
    dstp = jnp.concatenate([dst, N + jnp.arange(pad, dtype=i32) % (_NPAD - N)])
    src3 = srcp.reshape(16, _NCHUNK, _CHUNK)
    src_pair = jnp.stack([2 * src3, 2 * src3 + 1])
    dst3 = dstp.reshape(16, _NCHUNK, _CHUNK)
    xcat = x.reshape(2 * N, _DH)
    zeros_blk = jnp.zeros((_RPT, _DH), f32)

    mesh = plsc.VectorSubcoreMesh(core_axis_name="c", subcore_axis_name="s")
    run = functools.partial(
        pl.kernel,
        mesh=mesh,
        out_type=jax.ShapeDtypeStruct((2, _NPAD, _DH), f32),
        scratch_types=[
            pltpu.VMEM((_NCHUNK, _CHUNK), i32),
            pltpu.VMEM((_NCHUNK, _CHUNK), i32),
            pltpu.VMEM((_CHUNK, _DH), f32),
            pltpu.VMEM_SHARED((_NPAD, _DH), f32),
            pltpu.SemaphoreType.DMA,
        ],
    )(_sc_body)
    return run(xcat, src_pair, dst3, zeros_blk)


def _tc2_body(m_ref, b_ref, alo_ref, ahi_ref, num_ref, den_ref, mx_ref,
              wbb_ref, wc2b_ref, wb2c_ref, wcc_ref, wp_ref, s_ref, p_ref,
              out_ref, cflat_s, cemb_s, bp_s, cnt_s):
    i = pl.program_id(0)
    f32 = jnp.float32
    bf16 = jnp.bfloat16

    @pl.when(i == 0)
    def _():
        cflat = num_ref[...] / (den_ref[...] + 1e-6)
        cflat_s[...] = cflat.astype(bf16)
        b2c = jnp.dot(mx_ref[...].astype(bf16), wb2c_ref[...],
                      preferred_element_type=f32)
        c2c = jnp.dot(jnp.dot(s_ref[...], cflat.astype(bf16),
                              preferred_element_type=f32).astype(bf16),
                      wcc_ref[...], preferred_element_type=f32)
        cemb_s[...] = jnp.maximum(b2c + c2c, 0.0).astype(bf16)
        bp_s[...] = jnp.zeros_like(bp_s)
        cnt_s[...] = jnp.zeros_like(cnt_s)

    Mb = m_ref[...]
    c2b = jnp.dot(jnp.dot(Mb, cflat_s[...], preferred_element_type=f32).astype(bf16),
                  wc2b_ref[...], preferred_element_type=f32)
    wbb = wbb_ref[...]
    agg_wbb = (jnp.dot(alo_ref[0].astype(bf16), wbb[:_DH, :],
                       preferred_element_type=f32)
               + jnp.dot(ahi_ref[0].astype(bf16), wbb[_DH:, :],
                         preferred_element_type=f32))
    base_emb = jnp.maximum(agg_wbb + c2b, 0.0)

    lane16 = lax.broadcasted_iota(jnp.int32, (_BN, G), 1)
    oh = (b_ref[...] == lane16).astype(bf16)
    dn = (((0,), (0,)), ((), ()))
    bp_s[...] += lax.dot_general(oh, base_emb.astype(bf16), dn,
                                 preferred_element_type=f32)
    cnt_s[...] += lax.dot_general(oh, jnp.ones_like(base_emb, bf16), dn,
                                  preferred_element_type=f32)

    @pl.when(i == _GRID - 1)
    def _():
        base_pool = bp_s[...] / (cnt_s[...] + 1e-6)
        cent_pool = jnp.dot(p_ref[...], cemb_s[...], preferred_element_type=f32)
        wp = wp_ref[...]
        out_ref[...] = (jnp.dot(base_pool.astype(bf16), wp[:D, :],
                                preferred_element_type=f32)
                        + jnp.dot(cent_pool.astype(bf16), wp[D:, :],
                                  preferred_element_type=f32))


def _tc2(M, batch_col, agg_pair, num, den, mx,
         W_bb, W_c2b, W_b2c, W_cc, W_pred, S, P):
    f32 = jnp.float32
    return pl.pallas_call(
        _tc2_body,
        grid=(_GRID,),
        in_specs=[
            pl.BlockSpec((_BN, GC), lambda i: (i, 0)),
            pl.BlockSpec((_BN, 1), lambda i: (i, 0)),
            pl.BlockSpec((1, _BN, _DH), lambda i: (0, i, 0)),
            pl.BlockSpec((1, _BN, _DH), lambda i: (1, i, 0)),
            pl.BlockSpec((GC, D), lambda i: (0, 0)),
            pl.BlockSpec((GC, D), lambda i: (0, 0)),
            pl.BlockSpec((GC, D), lambda i: (0, 0)),
            pl.BlockSpec((D, D), lambda i: (0, 0)),
            pl.BlockSpec((D, D), lambda i: (0, 0)),
            pl.BlockSpec((D, D), lambda i: (0, 0)),
            pl.BlockSpec((D, D), lambda i: (0, 0)),
            pl.BlockSpec((2 * D, GC), lambda i: (0, 0)),
            pl.BlockSpec((GC, GC), lambda i: (0, 0)),
            pl.BlockSpec((G, GC), lambda i: (0, 0)),
        ],
        out_specs=pl.BlockSpec((G, GC), lambda i: (0, 0)),
        out_shape=jax.ShapeDtypeStruct((G, GC), f32),
        scratch_shapes=[
            pltpu.VMEM((GC, D), jnp.bfloat16),
            pltpu.VMEM((GC, D), jnp.bfloat16),
            pltpu.VMEM((G, D), f32),
            pltpu.VMEM((G, D), f32),
        ],
    )(M, batch_col, agg_pair, agg_pair, num, den, mx,
      W_bb, W_c2b, W_b2c, W_cc, W_pred, S, P)


def kernel(x, edge_index, batch, W_s1, W_s2, W_bc, W_bb, W_b2c, W_c2b, W_cc, W_pred):
    f32 = jnp.float32
    bf16 = jnp.bfloat16
    batch_col = batch.astype(jnp.int32).reshape(N, 1)
    W_s2p = jnp.pad(W_s2, ((0, 0), (0, GC - C))).astype(bf16)
    j = jnp.arange(GC)
    T = (j[:, None] == (j[None, :] % C)).astype(bf16)
    S = ((j[:, None] // C == j[None, :] // C).astype(f32)
         - jnp.eye(GC, dtype=f32)).astype(bf16)
    P = ((jnp.arange(G)[:, None] == (j[None, :] // C)).astype(f32)
         / C).astype(bf16)

    agg_pair = _sc_segment_sum(x, edge_index)
    M, num, den, mx = _tc1(x, batch_col, W_s1.astype(bf16), W_s2p,
                           W_bc.astype(bf16), T)
    return _tc2(M, batch_col, agg_pair, num, den, mx,
                W_bb.astype(bf16), W_c2b.astype(bf16), W_b2c.astype(bf16),
                W_cc.astype(bf16), W_pred.astype(bf16), S, P)


# speedup vs baseline: 1.0182x; 1.0182x over previous
import functools

import jax
import jax.numpy as jnp
from jax import lax
from jax.experimental import pallas as pl
from jax.experimental.pallas import tpu as pltpu
from jax.experimental.pallas import tpu_sc as plsc

N = 10000
E = 160000
D = 256
C = 8
G = 16
GC = G * C


_BN = 2000
_GRID = N // _BN


def _tc1_body(x_ref, b_ref, ws1_ref, ws2_ref, wbc_ref, t_ref,
              m_ref, num_ref, den_ref, mx_ref):
    i = pl.program_id(0)
    f32 = jnp.float32
    bf16 = jnp.bfloat16
    x16 = x_ref[...].astype(bf16)
    h = jnp.maximum(jnp.dot(x16, ws1_ref[...], preferred_element_type=f32), 0.0)
    sc = jnp.dot(h.astype(bf16), ws2_ref[...], preferred_element_type=f32)
    lane = lax.broadcasted_iota(jnp.int32, sc.shape, 1)
    scm = jnp.where(lane < C, sc, jnp.float32(-1e30))
    scm = scm - jnp.max(scm, axis=1, keepdims=True)
    e = jnp.exp(scm)
    sm = e / jnp.sum(e, axis=1, keepdims=True)
    m_tiled = jnp.dot(sm.astype(bf16), t_ref[...], preferred_element_type=f32)
    oh_rep = (b_ref[...] == (lane // C)).astype(f32)
    M16 = (oh_rep * m_tiled).astype(bf16)
    m_ref[...] = M16

    h2 = jnp.dot(x16, wbc_ref[...], preferred_element_type=f32)
    dn = (((0,), (0,)), ((), ()))
    num_p = lax.dot_general(M16, h2.astype(bf16), dn, preferred_element_type=f32)
    den_p = lax.dot_general(M16, jnp.ones_like(h2, bf16), dn,
                            preferred_element_type=f32)
    mx_p = lax.dot_general(M16, x16, dn, preferred_element_type=f32)

    @pl.when(i == 0)
    def _():
        num_ref[...] = num_p
        den_ref[...] = den_p
        mx_ref[...] = mx_p

    @pl.when(i > 0)
    def _():
        num_ref[...] += num_p
        den_ref[...] += den_p
        mx_ref[...] += mx_p


def _tc1(x, batch_col, W_s1, W_s2p, W_bc, T):
    f32 = jnp.float32
    return pl.pallas_call(
        _tc1_body,
        grid=(_GRID,),
        in_specs=[
            pl.BlockSpec((_BN, D), lambda i: (i, 0)),
            pl.BlockSpec((_BN, 1), lambda i: (i, 0)),
            pl.BlockSpec((D, D), lambda i: (0, 0)),
            pl.BlockSpec((D, GC), lambda i: (0, 0)),
            pl.BlockSpec((D, D), lambda i: (0, 0)),
            pl.BlockSpec((GC, GC), lambda i: (0, 0)),
        ],
        out_specs=[
            pl.BlockSpec((_BN, GC), lambda i: (i, 0)),
            pl.BlockSpec((GC, D), lambda i: (0, 0)),
            pl.BlockSpec((GC, D), lambda i: (0, 0)),
            pl.BlockSpec((GC, D), lambda i: (0, 0)),
        ],
        out_shape=[
            jax.ShapeDtypeStruct((N, GC), jnp.bfloat16),
            jax.ShapeDtypeStruct((GC, D), f32),
            jax.ShapeDtypeStruct((GC, D), f32),
            jax.ShapeDtypeStruct((GC, D), f32),
        ],
    )(x, batch_col, W_s1, W_s2p, W_bc, T)


_DH = 128
_CHUNK = 128
_NCHUNK = 79
_EPT = _CHUNK * _NCHUNK
_EPAD = 16 * _EPT
_RPT = 632
_NPAD = 16 * _RPT


def _sc_body(xcat_hbm, src_hbm, dst_hbm, zeros_hbm, out_hbm,
             src_v, dst_v, rows_v, acc_sh, sem):
    c = lax.axis_index("c")
    s = lax.axis_index("s")
    pltpu.sync_copy(zeros_hbm, acc_sh.at[pl.ds(s * _RPT, _RPT)])
    pltpu.sync_copy(src_hbm.at[c, s], src_v)
    pltpu.sync_copy(dst_hbm.at[s], dst_v)
    plsc.subcore_barrier()

    def body(j, carry):
        pltpu.async_copy(xcat_hbm.at[src_v.at[j]], rows_v, sem).wait()
        pltpu.sync_copy(rows_v, acc_sh.at[dst_v.at[j]], add=True)
        return carry

    lax.fori_loop(0, _NCHUNK, body, 0)
    plsc.subcore_barrier()
    pltpu.sync_copy(acc_sh.at[pl.ds(s * _RPT, _RPT)],
                    out_hbm.at[pl.ds(c * _NPAD + s * _RPT, _RPT)])


def _sc_segment_sum(x, edge_index):
    f32 = jnp.float32
    i32 = jnp.int32
    src = edge_index[0].astype(i32)
    dst = edge_index[1].astype(i32)
    pad = _EPAD - E
    srcp = jnp.concatenate([src, jnp.full((pad,), N, i32)])
    dstp = jnp.concatenate([dst, jnp.arange(pad, dtype=i32) % N])
    src3 = srcp.reshape(16, _NCHUNK, _CHUNK)
    src_pair = jnp.stack([src3, src3 + (N + 1)])
    dst3 = dstp.reshape(16, _NCHUNK, _CHUNK)
    xz = jnp.concatenate([x, jnp.zeros((1, D), f32)], axis=0)
    xcat = jnp.concatenate([xz[:, :_DH], xz[:, _DH:]], axis=0)
    zeros_blk = jnp.zeros((_RPT, _DH), f32)

    mesh = plsc.VectorSubcoreMesh(core_axis_name="c", subcore_axis_name="s")
    run = functools.partial(
        pl.kernel,
        mesh=mesh,
        out_type=jax.ShapeDtypeStruct((2 * _NPAD, _DH), f32),
        scratch_types=[
            pltpu.VMEM((_NCHUNK, _CHUNK), i32),
            pltpu.VMEM((_NCHUNK, _CHUNK), i32),
            pltpu.VMEM((_CHUNK, _DH), f32),
            pltpu.VMEM_SHARED((_NPAD, _DH), f32),
            pltpu.SemaphoreType.DMA,
        ],
    )(_sc_body)
    return run(xcat, src_pair, dst3, zeros_blk)


def _tc2_body(m_ref, b_ref, alo_ref, ahi_ref, num_ref, den_ref, mx_ref,
              wbb_ref, wc2b_ref, wb2c_ref, wcc_ref, wp_ref, s_ref, p_ref,
              out_ref, cflat_s, cemb_s, bp_s, cnt_s):
    i = pl.program_id(0)
    f32 = jnp.float32
    bf16 = jnp.bfloat16

    @pl.when(i == 0)
    def _():
        cflat = num_ref[...] / (den_ref[...] + 1e-6)
        cflat_s[...] = cflat.astype(bf16)
        b2c = jnp.dot(mx_ref[...].astype(bf16), wb2c_ref[...],
                      preferred_element_type=f32)
        c2c = jnp.dot(jnp.dot(s_ref[...], cflat.astype(bf16),
                              preferred_element_type=f32).astype(bf16),
                      wcc_ref[...], preferred_element_type=f32)
        cemb_s[...] = jnp.maximum(b2c + c2c, 0.0).astype(bf16)
        bp_s[...] = jnp.zeros_like(bp_s)
        cnt_s[...] = jnp.zeros_like(cnt_s)

    Mb = m_ref[...]
    c2b = jnp.dot(jnp.dot(Mb, cflat_s[...], preferred_element_type=f32).astype(bf16),
                  wc2b_ref[...], preferred_element_type=f32)
    wbb = wbb_ref[...]
    agg_wbb = (jnp.dot(alo_ref[0].astype(bf16), wbb[:_DH, :],
                       preferred_element_type=f32)
               + jnp.dot(ahi_ref[0].astype(bf16), wbb[_DH:, :],
                         preferred_element_type=f32))
    base_emb = jnp.maximum(agg_wbb + c2b, 0.0)

    lane16 = lax.broadcasted_iota(jnp.int32, (_BN, G), 1)
    oh = (b_ref[...] == lane16).astype(bf16)
    dn = (((0,), (0,)), ((), ()))
    bp_s[...] += lax.dot_general(oh, base_emb.astype(bf16), dn,
                                 preferred_element_type=f32)
    cnt_s[...] += lax.dot_general(oh, jnp.ones_like(base_emb, bf16), dn,
                                  preferred_element_type=f32)

    @pl.when(i == _GRID - 1)
    def _():
        base_pool = bp_s[...] / (cnt_s[...] + 1e-6)
        cent_pool = jnp.dot(p_ref[...], cemb_s[...], preferred_element_type=f32)
        wp = wp_ref[...]
        out_ref[...] = (jnp.dot(base_pool.astype(bf16), wp[:D, :],
                                preferred_element_type=f32)
                        + jnp.dot(cent_pool.astype(bf16), wp[D:, :],
                                  preferred_element_type=f32))


def _tc2(M, batch_col, agg_pair, num, den, mx,
         W_bb, W_c2b, W_b2c, W_cc, W_pred, S, P):
    f32 = jnp.float32
    return pl.pallas_call(
        _tc2_body,
        grid=(_GRID,),
        in_specs=[
            pl.BlockSpec((_BN, GC), lambda i: (i, 0)),
            pl.BlockSpec((_BN, 1), lambda i: (i, 0)),
            pl.BlockSpec((1, _BN, _DH), lambda i: (0, i, 0)),
            pl.BlockSpec((1, _BN, _DH), lambda i: (1, i, 0)),
            pl.BlockSpec((GC, D), lambda i: (0, 0)),
            pl.BlockSpec((GC, D), lambda i: (0, 0)),
            pl.BlockSpec((GC, D), lambda i: (0, 0)),
            pl.BlockSpec((D, D), lambda i: (0, 0)),
            pl.BlockSpec((D, D), lambda i: (0, 0)),
            pl.BlockSpec((D, D), lambda i: (0, 0)),
            pl.BlockSpec((D, D), lambda i: (0, 0)),
            pl.BlockSpec((2 * D, GC), lambda i: (0, 0)),
            pl.BlockSpec((GC, GC), lambda i: (0, 0)),
            pl.BlockSpec((G, GC), lambda i: (0, 0)),
        ],
        out_specs=pl.BlockSpec((G, GC), lambda i: (0, 0)),
        out_shape=jax.ShapeDtypeStruct((G, GC), f32),
        scratch_shapes=[
            pltpu.VMEM((GC, D), jnp.bfloat16),
            pltpu.VMEM((GC, D), jnp.bfloat16),
            pltpu.VMEM((G, D), f32),
            pltpu.VMEM((G, D), f32),
        ],
    )(M, batch_col, agg_pair, agg_pair, num, den, mx,
      W_bb, W_c2b, W_b2c, W_cc, W_pred, S, P)


def kernel(x, edge_index, batch, W_s1, W_s2, W_bc, W_bb, W_b2c, W_c2b, W_cc, W_pred):
    f32 = jnp.float32
    bf16 = jnp.bfloat16
    batch_col = batch.astype(jnp.int32).reshape(N, 1)
    W_s2p = jnp.pad(W_s2, ((0, 0), (0, GC - C))).astype(bf16)
    j = jnp.arange(GC)
    T = (j[:, None] == (j[None, :] % C)).astype(bf16)
    S = ((j[:, None] // C == j[None, :] // C).astype(f32)
         - jnp.eye(GC, dtype=f32)).astype(bf16)
    P = ((jnp.arange(G)[:, None] == (j[None, :] // C)).astype(f32)
         / C).astype(bf16)

    agg = _sc_segment_sum(x, edge_index)
    agg_pair = jnp.stack([agg[:N], agg[_NPAD:_NPAD + N]])
    M, num, den, mx = _tc1(x, batch_col, W_s1.astype(bf16), W_s2p,
                           W_bc.astype(bf16), T)
    return _tc2(M, batch_col, agg_pair, num, den, mx,
                W_bb.astype(bf16), W_c2b.astype(bf16), W_b2c.astype(bf16),
                W_cc.astype(bf16), W_pred.astype(bf16), S, P)
